# Initial kernel scaffold; baseline (speedup 1.0000x reference)
import jax
import jax.numpy as jnp
from jax.experimental import pallas as pl


def kernel(depth, extrinsics, intrinsics, volume, origin, resolution, weights):
    raise NotImplementedError("write your pallas kernel here")


# speedup vs baseline: 1.0993x; 1.0993x over previous
import functools

import jax
import jax.numpy as jnp
from jax import lax
from jax.experimental import pallas as pl
from jax.experimental.pallas import tpu as pltpu
from jax.experimental.pallas import tpu_sc as plsc

_H, _W = 240, 320
_NPIX = _H * _W
_NSAMP = 9
_NWORK = 32
_PW = _NPIX // _NWORK
_CP = 160
_NCHUNK = _PW // _CP
_NG = _CP // 16
_SAMP = _CP * _NSAMP
_IDXN = _SAMP * 8
_ROWS = _IDXN // 128


def _rsqrt(x):
    i = lax.bitcast_convert_type(x, jnp.int32)
    i = jnp.int32(0x5F3759DF) - lax.shift_right_logical(i, 1)
    y = lax.bitcast_convert_type(i, jnp.float32)
    for _ in range(3):
        y = y * (1.5 - 0.5 * x * y * y)
    return y


def _b16(x):
    i = lax.bitcast_convert_type(x, jnp.int32)
    r = i + jnp.int32(0x7FFF) + lax.bitwise_and(lax.shift_right_logical(i, 16),
                                                jnp.int32(1))
    r = lax.bitwise_and(r, jnp.int32(-65536))
    return lax.bitcast_convert_type(r, jnp.float32)


def _floor(p):
    ti = p.astype(jnp.int32)
    tf = ti.astype(jnp.float32)
    over = tf > p
    return jnp.where(over, tf - 1.0, tf), jnp.where(over, ti - 1, ti)


def _sc_body(depth_hbm, const_hbm, vol_hbm, wvol_hbm,
             fv_hbm, fw_hbm, rp_hbm, ind_hbm, w_hbm, co_hbm,
             depth_v, const_v, co_v, rp_v, ind_v, w_v, wm_v, idx_v,
             gv_v, gw_v, fv_v, fw_v, sem):
    cid = lax.axis_index("c")
    sid = lax.axis_index("s")
    wid = sid * 2 + cid
    pltpu.sync_copy(const_hbm, const_v)

    def crow(r):
        return const_v[pl.ds(r * 16, 16)]

    @pl.loop(0, _NCHUNK)
    def _chunk(chunk):
        pixbase = pl.multiple_of(wid * _PW + chunk * _CP, 32)
        pltpu.sync_copy(depth_hbm.at[pl.ds(pixbase, _CP)], depth_v)

        k00, k01, k02 = _b16(crow(0)), _b16(crow(1)), _b16(crow(2))
        k10, k11, k12 = _b16(crow(3)), _b16(crow(4)), _b16(crow(5))
        k20, k21, k22 = _b16(crow(6)), _b16(crow(7)), _b16(crow(8))
        e00, e01, e02, e03 = _b16(crow(9)), _b16(crow(10)), _b16(crow(11)), _b16(crow(12))
        e10, e11, e12, e13 = _b16(crow(13)), _b16(crow(14)), _b16(crow(15)), _b16(crow(16))
        e20, e21, e22, e23 = _b16(crow(17)), _b16(crow(18)), _b16(crow(19)), _b16(crow(20))
        t0, t1, t2 = crow(12), crow(16), crow(20)
        o0, o1, o2 = crow(21), crow(22), crow(23)
        res = crow(24)
        eye0 = (t0 - o0) / res
        eye1 = (t1 - o1) / res
        eye2 = (t2 - o2) / res

        @pl.loop(0, _NG)
        def _grp(g):
            lane = lax.iota(jnp.int32, 16)
            ploc = g * 16 + lane
            m = wid * _NCHUNK + chunk
            irow = lax.shift_right_arithmetic(m, 1)
            jcol0 = jnp.bitwise_and(m, 1) * (_W // 2) + g * 16
            zero16 = jnp.zeros((16,), jnp.float32)
            fi = irow.astype(jnp.float32) + zero16
            fj = jcol0.astype(jnp.float32) + lane.astype(jnp.float32)
            z = depth_v[pl.ds(g * 16, 16)]
            pp0 = _b16(fj * z)
            pp1 = _b16(fi * z)
            pp2 = _b16(z)
            pc0 = (k00 * pp0 + k01 * pp1) + k02 * pp2
            pc1 = (k10 * pp0 + k11 * pp1) + k12 * pp2
            pc2 = (k20 * pp0 + k21 * pp1) + k22 * pp2
            pb0, pb1, pb2 = _b16(pc0), _b16(pc1), _b16(pc2)
            c0 = ((e00 * pb0 + e01 * pb1) + e02 * pb2) + e03
            c1 = ((e10 * pb0 + e11 * pb1) + e12 * pb2) + e13
            c2 = ((e20 * pb0 + e21 * pb1) + e22 * pb2) + e23
            d3 = ploc * 3
            plsc.store_scatter(co_v, [d3], c0)
            plsc.store_scatter(co_v, [d3 + 1], c1)
            plsc.store_scatter(co_v, [d3 + 2], c2)
            cen0 = (c0 - o0) / res
            cen1 = (c1 - o1) / res
            cen2 = (c2 - o2) / res
            dx = cen0 - eye0
            dy = cen1 - eye1
            dz = cen2 - eye2
            nn = dx * dx + dy * dy + dz * dz
            sq = nn * _rsqrt(nn)
            sq = 0.5 * (sq + nn / jnp.maximum(sq, 1e-30))
            sq = jnp.maximum(sq, 1e-12)
            ux, uy, uz = dx / sq, dy / sq, dz / sq
            for s in range(_NSAMP):
                sf = jnp.float32(float(s - 4))
                p0 = cen0 + sf * ux
                p1 = cen1 + sf * uy
                p2 = cen2 + sf * uz
                b27 = ploc * 27 + (s * 3)
                plsc.store_scatter(rp_v, [b27], p0)
                plsc.store_scatter(rp_v, [b27 + 1], p1)
                plsc.store_scatter(rp_v, [b27 + 2], p2)
                f0, i0 = _floor(p0)
                f1, i1 = _floor(p1)
                f2, i2 = _floor(p2)
                df0 = jnp.abs(p0 - (f0 + 0.5))
                df1 = jnp.abs(p1 - (f1 + 0.5))
                df2 = jnp.abs(p2 - (f2 + 0.5))
                X = (1.0 - df0, df0)
                Y = (1.0 - df1, df1)
                Z = (1.0 - df2, df2)
                sampv = ploc * 9 + s
                b216 = ploc * 216 + (s * 24)
                b72 = ploc * 72 + (s * 8)
                for k in range(8):
                    bx, by, bz = (k >> 2) & 1, (k >> 1) & 1, k & 1
                    wgt = X[bx] * Y[by] * Z[bz]
                    plsc.store_scatter(ind_v, [b216 + (k * 3)], f0 + float(bx))
                    plsc.store_scatter(ind_v, [b216 + (k * 3 + 1)], f1 + float(by))
                    plsc.store_scatter(ind_v, [b216 + (k * 3 + 2)], f2 + float(bz))
                    plsc.store_scatter(w_v, [b72 + k], wgt)
                    q0 = i0 + bx
                    q1 = i1 + by
                    q2 = i2 + bz
                    valid = ((q0 >= 0) & (q0 < 256) & (q1 >= 0) & (q1 < 256)
                             & (q2 >= 0) & (q2 < 256))
                    g0 = jnp.clip(q0, 0, 255)
                    g1 = jnp.clip(q1, 0, 255)
                    g2 = jnp.clip(q2, 0, 255)
                    flat = (g0 * 256 + g1) * 256 + g2
                    wm = jnp.where(valid, wgt, 0.0)
                    pos = k * _SAMP + sampv
                    plsc.store_scatter(wm_v, [pos], wm)
                    plsc.store_scatter(idx_v, [pos], flat)

        @pl.loop(0, _ROWS)
        def _fire(r):
            off = pl.multiple_of(r * 128, 128)
            isl = idx_v.at[pl.ds(off, 128)]
            pltpu.async_copy(vol_hbm.at[isl], gv_v.at[pl.ds(off, 128)], sem)
            pltpu.async_copy(wvol_hbm.at[isl], gw_v.at[pl.ds(off, 128)], sem)

        @pl.loop(0, _ROWS)
        def _drain(r):
            off = pl.multiple_of(r * 128, 128)
            isl = idx_v.at[pl.ds(off, 128)]
            pltpu.make_async_copy(vol_hbm.at[isl],
                                  gv_v.at[pl.ds(off, 128)], sem).wait()
            pltpu.make_async_copy(wvol_hbm.at[isl],
                                  gw_v.at[pl.ds(off, 128)], sem).wait()

        @pl.loop(0, _SAMP // 16)
        def _comb(tv):
            off = pl.multiple_of(tv * 16, 16)
            av = jnp.zeros((16,), jnp.float32)
            aw = jnp.zeros((16,), jnp.float32)
            for k in range(8):
                wmk = wm_v[pl.ds(k * _SAMP + off, 16)]
                av = av + wmk * gv_v[pl.ds(k * _SAMP + off, 16)]
                aw = aw + wmk * gw_v[pl.ds(k * _SAMP + off, 16)]
            fv_v[pl.ds(off, 16)] = av
            fw_v[pl.ds(off, 16)] = aw

        pltpu.sync_copy(fv_v, fv_hbm.at[pl.ds(pl.multiple_of(pixbase * 9, 8), _SAMP)])
        pltpu.sync_copy(fw_v, fw_hbm.at[pl.ds(pl.multiple_of(pixbase * 9, 8), _SAMP)])
        pltpu.sync_copy(rp_v, rp_hbm.at[pl.ds(pl.multiple_of(pixbase * 27, 8), _CP * 27)])
        pltpu.sync_copy(ind_v, ind_hbm.at[pl.ds(pl.multiple_of(pixbase * 216, 8), _CP * 216)])
        pltpu.sync_copy(w_v, w_hbm.at[pl.ds(pl.multiple_of(pixbase * 72, 8), _CP * 72)])
        pltpu.sync_copy(co_v, co_hbm.at[pl.ds(pl.multiple_of(pixbase * 3, 8), _CP * 3)])


@jax.jit
def _sc_run(depth_flat, const256, vol_flat, wvol_flat):
    f32 = jnp.float32
    run = pl.kernel(
        _sc_body,
        out_type=[
            jax.ShapeDtypeStruct((_NPIX * _NSAMP,), f32),
            jax.ShapeDtypeStruct((_NPIX * _NSAMP,), f32),
            jax.ShapeDtypeStruct((_NPIX * _NSAMP * 3,), f32),
            jax.ShapeDtypeStruct((_NPIX * _NSAMP * 24,), f32),
            jax.ShapeDtypeStruct((_NPIX * _NSAMP * 8,), f32),
            jax.ShapeDtypeStruct((_NPIX * 3,), f32),
        ],
        mesh=plsc.VectorSubcoreMesh(core_axis_name="c", subcore_axis_name="s",
                                    num_cores=2, num_subcores=16),
        compiler_params=pltpu.CompilerParams(needs_layout_passes=False),
        scratch_types=[
            pltpu.VMEM((_CP,), f32),
            pltpu.VMEM((400,), f32),
            pltpu.VMEM((_CP * 3,), f32),
            pltpu.VMEM((_CP * 27,), f32),
            pltpu.VMEM((_CP * 216,), f32),
            pltpu.VMEM((_CP * 72,), f32),
            pltpu.VMEM((_IDXN,), f32),
            pltpu.VMEM((_IDXN,), jnp.int32),
            pltpu.VMEM((_IDXN,), f32),
            pltpu.VMEM((_IDXN,), f32),
            pltpu.VMEM((_SAMP,), f32),
            pltpu.VMEM((_SAMP,), f32),
            pltpu.SemaphoreType.DMA,
        ],
    )
    return run(depth_flat, const256, vol_flat, wvol_flat)


def kernel(depth, extrinsics, intrinsics, volume, origin, resolution, weights):
    b, h, w = depth.shape
    f32 = jnp.float32
    Kinv = jnp.linalg.inv(intrinsics.astype(f32))[0]
    E = extrinsics[0].astype(f32)
    res = jnp.asarray(resolution, f32).reshape(1)
    c25 = jnp.concatenate([Kinv.reshape(-1), E[:3, :4].reshape(-1),
                           origin.astype(f32), res])
    const400 = jnp.repeat(c25, 16)

    fv, fw, rp, ind, wout, co = _sc_run(
        depth.reshape(-1), const400, volume.reshape(-1), weights.reshape(-1))

    return (fv.reshape(b, _NPIX, _NSAMP),
            fw.reshape(b, _NPIX, _NSAMP),
            rp.reshape(b, _NPIX, _NSAMP, 3),
            depth.reshape(b, h * w),
            ind.reshape(b, _NPIX, _NSAMP, 8, 3),
            wout.reshape(b, _NPIX, _NSAMP, 8),
            co.reshape(b, _NPIX, 3))
